# baseline (speedup 1.0000x reference)
import jax
import jax.numpy as jnp
from jax import lax
from jax.experimental import pallas as pl
from jax.experimental.pallas import tpu as pltpu
from jax.experimental.pallas import tpu_sc as plsc

NC, NS = 2, 16
N, E, D, DE = 10000, 320000, 128, 16
EPC = E // NC
EPT = EPC // NS
KS = 80
NCHUNKS = EPT // KS
KG = 200
NCHUNKG = EPT // KG
IB = 25
ZR = 624
ZREM = N - NS * ZR

_mesh = plsc.VectorSubcoreMesh(core_axis_name="c", subcore_axis_name="s",
                               num_cores=NC, num_subcores=NS)


def _row_ranges(s):
    r0 = s * ZR
    return [(r0 + i * KS, min(KS, ZR - i * KS))
            for i in range((ZR + KS - 1) // KS)]


EPT1 = E // NS
NCHUNK1 = EPT1 // KS


def _seg_loop(table, src1d, dst1d, acc, base_e, nblocks,
              sblk, dblk, d0, d1, g0, sem0, g1, sem1):
    def gidx(k):
        return sblk.at[pl.ds(k * KS, KS)]

    def prep(k, dref):
        for i in range(KS // 16):
            dref[pl.ds(i * 16, 16)] = dblk[pl.ds(k * KS + i * 16, 16)]

    def wait_gather(gb, sem):
        pltpu.make_async_copy(table.at[pl.ds(0, KS)], gb, sem).wait()

    def block(ib, carry):
        off = pl.multiple_of(base_e + ib * (IB * KS), 8)
        pltpu.sync_copy(src1d.at[pl.ds(off, IB * KS)], sblk)
        pltpu.sync_copy(dst1d.at[pl.ds(off, IB * KS)], dblk)
        pltpu.async_copy(table.at[gidx(0)], g0, sem0)
        prep(0, d0)

        def pair(jj, cc):
            b = 2 * jj + 1
            c2 = 2 * jj + 2
            pltpu.async_copy(table.at[gidx(b)], g1, sem1)
            prep(b, d1)
            wait_gather(g0, sem0)
            pltpu.sync_copy(g0, acc.at[d0], add=True)

            @pl.when(c2 < IB)
            def _():
                pltpu.async_copy(table.at[gidx(c2)], g0, sem0)
                prep(c2, d0)

            wait_gather(g1, sem1)
            pltpu.sync_copy(g1, acc.at[d1], add=True)
            return cc

        lax.fori_loop(0, IB // 2, pair, 0)
        if IB % 2 == 1:
            wait_gather(g0, sem0)
            pltpu.sync_copy(g0, acc.at[d0], add=True)
        return carry

    lax.fori_loop(0, nblocks, block, 0)


def _zero_acc(zkd, gbuf, acc, s):
    pltpu.sync_copy(zkd, gbuf)
    for off, sz in _row_ranges(s):
        pltpu.sync_copy(gbuf.at[pl.ds(0, sz)], acc.at[pl.ds(off, sz)])

    @pl.when(s == 0)
    def _():
        pltpu.sync_copy(gbuf.at[pl.ds(0, ZREM)], acc.at[pl.ds(NS * ZR, ZREM)])


def _agg_deg_body(table, src1d, dst1d, zkd, onesd, agg_out, deg_out,
                  sblk, dblk, d0, d1, g0, sem0, g1, sem1, acc):
    c = lax.axis_index("c")
    s = lax.axis_index("s")

    _zero_acc(zkd, g0, acc, s)

    @pl.when(c == 1)
    def _():
        pltpu.sync_copy(onesd, g0)

    plsc.subcore_barrier()

    base_e = s * EPT1

    @pl.when(c == 0)
    def _():
        _seg_loop(table, src1d, dst1d, acc, base_e, NCHUNK1 // IB,
                  sblk, dblk, d0, d1, g0, sem0, g1, sem1)

    @pl.when(c == 1)
    def _():
        def block(ib, carry):
            off = pl.multiple_of(base_e + ib * (IB * KS), 8)
            pltpu.sync_copy(dst1d.at[pl.ds(off, IB * KS)], dblk)

            def chunk(k, cc):
                for i in range(KS // 16):
                    d0[pl.ds(i * 16, 16)] = dblk[pl.ds(k * KS + i * 16, 16)]
                pltpu.sync_copy(g0, acc.at[d0], add=True)
                return cc

            lax.fori_loop(0, IB, chunk, 0)
            return carry

        lax.fori_loop(0, NCHUNK1 // IB, block, 0)

    plsc.subcore_barrier()

    r0 = s * ZR

    @pl.when(c == 0)
    def _():
        pltpu.sync_copy(acc.at[pl.ds(r0, ZR)], agg_out.at[pl.ds(r0, ZR)])

    @pl.when(c == 1)
    def _():
        pltpu.sync_copy(acc.at[pl.ds(r0, ZR)], deg_out.at[pl.ds(r0, ZR)])

    @pl.when((s == 0) & (c == 0))
    def _():
        pltpu.sync_copy(acc.at[pl.ds(NS * ZR, ZREM)],
                        agg_out.at[pl.ds(NS * ZR, ZREM)])

    @pl.when((s == 0) & (c == 1))
    def _():
        pltpu.sync_copy(acc.at[pl.ds(NS * ZR, ZREM)],
                        deg_out.at[pl.ds(NS * ZR, ZREM)])


_SEG_SCRATCH = (
    pltpu.VMEM((IB * KS,), jnp.int32),
    pltpu.VMEM((IB * KS,), jnp.int32),
    pltpu.VMEM((KS,), jnp.int32),
    pltpu.VMEM((KS,), jnp.int32),
    pltpu.VMEM((KS, D), jnp.float32),
    pltpu.SemaphoreType.DMA,
    pltpu.VMEM((KS, D), jnp.float32),
    pltpu.SemaphoreType.DMA,
    pltpu.VMEM_SHARED((N, D), jnp.float32),
)

_agg_deg = pl.kernel(
    _agg_deg_body,
    out_type=(jax.ShapeDtypeStruct((N, D), jnp.float32),
              jax.ShapeDtypeStruct((N, D), jnp.float32)),
    mesh=_mesh,
    scratch_types=_SEG_SCRATCH,
)


def _segsum_body(table, src1d, dst1d, zkd, agg_out,
                 sblk, dblk, d0, d1, g0, sem0, g1, sem1, acc):
    c = lax.axis_index("c")
    s = lax.axis_index("s")

    _zero_acc(zkd, g0, acc, s)
    plsc.subcore_barrier()

    base_e = (c * NS + s) * EPT
    _seg_loop(table, src1d, dst1d, acc, base_e, NCHUNKS // IB,
              sblk, dblk, d0, d1, g0, sem0, g1, sem1)

    plsc.subcore_barrier()

    out_base = c * N
    r0 = s * ZR
    pltpu.sync_copy(acc.at[pl.ds(r0, ZR)], agg_out.at[pl.ds(out_base + r0, ZR)])

    @pl.when(s == 0)
    def _():
        pltpu.sync_copy(acc.at[pl.ds(NS * ZR, ZREM)],
                        agg_out.at[pl.ds(out_base + NS * ZR, ZREM)])


_segsum = pl.kernel(
    _segsum_body,
    out_type=(jax.ShapeDtypeStruct((NC * N, D), jnp.float32),),
    mesh=_mesh,
    scratch_types=_SEG_SCRATCH,
)


def _gpair_body(A, B, u1d, v1d, g_out,
                ublk, vblk, r0, r1, r2, sa0, sa1, sa2, sb0, sb1, sb2):
    c = lax.axis_index("c")
    s = lax.axis_index("s")
    base_e = (c * NS + s) * EPT
    pltpu.sync_copy(u1d.at[pl.ds(base_e, EPT)], ublk)
    pltpu.sync_copy(v1d.at[pl.ds(base_e, EPT)], vblk)

    bufs = (r0, r1, r2)
    sas = (sa0, sa1, sa2)
    sbs = (sb0, sb1, sb2)

    def start_a(j, t):
        pltpu.async_copy(A.at[ublk.at[pl.ds(j * KG, KG)]], bufs[t], sas[t])

    def add_b(j, t):
        pltpu.make_async_copy(A.at[pl.ds(0, KG)], bufs[t], sas[t]).wait()
        pltpu.async_copy(B.at[vblk.at[pl.ds(j * KG, KG)]], bufs[t],
                         sbs[t], add=True)

    def finish(j, t):
        off = pl.multiple_of(base_e + j * KG, 8)
        pltpu.make_async_copy(B.at[pl.ds(0, KG)], bufs[t], sbs[t]).wait()
        pltpu.sync_copy(bufs[t], g_out.at[pl.ds(off, KG)])

    start_a(0, 0)
    start_a(1, 1)
    add_b(0, 0)

    def trio(m, carry):
        k = 3 * m
        for t in range(3):
            start_a(k + t + 2, (t + 2) % 3)
            add_b(k + t + 1, (t + 1) % 3)
            finish(k + t, t)
        return carry

    lax.fori_loop(0, (NCHUNKG - 2) // 3, trio, 0)
    add_b(NCHUNKG - 1, (NCHUNKG - 1) % 3)
    finish(NCHUNKG - 2, (NCHUNKG - 2) % 3)
    finish(NCHUNKG - 1, (NCHUNKG - 1) % 3)


_gpair = pl.kernel(
    _gpair_body,
    out_type=(jax.ShapeDtypeStruct((E, D), jnp.float32),),
    mesh=_mesh,
    scratch_types=(
        pltpu.VMEM((EPT,), jnp.int32),
        pltpu.VMEM((EPT,), jnp.int32),
        pltpu.VMEM((KG, D), jnp.float32),
        pltpu.VMEM((KG, D), jnp.float32),
        pltpu.VMEM((KG, D), jnp.float32),
        pltpu.SemaphoreType.DMA,
        pltpu.SemaphoreType.DMA,
        pltpu.SemaphoreType.DMA,
        pltpu.SemaphoreType.DMA,
        pltpu.SemaphoreType.DMA,
        pltpu.SemaphoreType.DMA,
    ),
)


def _dt(p, w):
    return lax.dot_general(p, w, (((1,), (1,)), ((), ())),
                           preferred_element_type=jnp.float32)


BR = 1000


def _tc1_body(a, dg, xr, wl, bl, wr, out):
    mean = a[...] / jnp.maximum(dg[:, 0:1], 1.0)
    out[...] = jnp.maximum(_dt(mean, wl[...]) + _dt(xr[...], wr[...])
                           + bl[...], 0.0)


def _tc2_body(a0, a1, dg, hr, wl, bl, wr, wu, wv, aout, bout):
    mean = (a0[...] + a1[...]) / jnp.maximum(dg[:, 0:1], 1.0)
    h = jnp.maximum(_dt(mean, wl[...]) + _dt(hr[...], wr[...]) + bl[...], 0.0)
    aout[...] = _dt(h, wu[...])
    bout[...] = _dt(h, wv[...])


def _tc_layer1(agg, deg, x, wl, bl, wr):
    return pl.pallas_call(
        _tc1_body,
        grid=(N // BR,),
        in_specs=[
            pl.BlockSpec((BR, D), lambda i: (i, 0)),
            pl.BlockSpec((BR, D), lambda i: (i, 0)),
            pl.BlockSpec((BR, D), lambda i: (i, 0)),
            pl.BlockSpec((D, D), lambda i: (0, 0)),
            pl.BlockSpec((1, D), lambda i: (0, 0)),
            pl.BlockSpec((D, D), lambda i: (0, 0)),
        ],
        out_specs=pl.BlockSpec((BR, D), lambda i: (i, 0)),
        out_shape=jax.ShapeDtypeStruct((N, D), jnp.float32),
    )(agg, deg, x, wl, bl, wr)


def _tc_layer2(aggp, deg, h1, wl, bl, wr, wu, wv):
    return pl.pallas_call(
        _tc2_body,
        grid=(N // BR,),
        in_specs=[
            pl.BlockSpec((BR, D), lambda i: (i, 0)),
            pl.BlockSpec((BR, D), lambda i: (i + N // BR, 0)),
            pl.BlockSpec((BR, D), lambda i: (i, 0)),
            pl.BlockSpec((BR, D), lambda i: (i, 0)),
            pl.BlockSpec((D, D), lambda i: (0, 0)),
            pl.BlockSpec((1, D), lambda i: (0, 0)),
            pl.BlockSpec((D, D), lambda i: (0, 0)),
            pl.BlockSpec((D, D), lambda i: (0, 0)),
            pl.BlockSpec((D, D), lambda i: (0, 1)),
        ],
        out_specs=[pl.BlockSpec((BR, D), lambda i: (i, 0)),
                   pl.BlockSpec((BR, D), lambda i: (i, 0))],
        out_shape=[jax.ShapeDtypeStruct((N, D), jnp.float32),
                   jax.ShapeDtypeStruct((N, D), jnp.float32)],
    )(aggp, aggp, deg, h1, wl, bl, wr, wu, wv)


BE = 16000


RB = BE // 128


def _tc3_body(g, at, le, w1e, b1, w2, b2, out):
    hmid = jnp.maximum(g[...] + _dt(at[...], w1e[...]) + b1[...], 0.0)
    h3 = hmid.reshape(RB, 128, D)
    s = lax.dot_general(h3, w2[...], (((2,), (0,)), ((), ())),
                        preferred_element_type=jnp.float32)
    out[...] = (s + le[0] + b2[0]).reshape(1, RB, 128)


def _tc_edge(g, attr, le, w1e, b1, w2, b2):
    return pl.pallas_call(
        _tc3_body,
        grid=(E // BE,),
        in_specs=[
            pl.BlockSpec((BE, D), lambda i: (i, 0)),
            pl.BlockSpec((BE, DE), lambda i: (i, 0)),
            pl.BlockSpec((1, RB, 128), lambda i: (i, 0, 0)),
            pl.BlockSpec((D, DE), lambda i: (0, 0)),
            pl.BlockSpec((1, D), lambda i: (0, 0)),
            pl.BlockSpec((D,), lambda i: (0,)),
            pl.BlockSpec(memory_space=pltpu.SMEM),
        ],
        out_specs=pl.BlockSpec((1, RB, 128), lambda i: (i, 0, 0)),
        out_shape=jax.ShapeDtypeStruct((E // BE, RB, 128), jnp.float32),
    )(g, attr, le, w1e, b1, w2, b2)


def kernel(x, edge_index, edge_u, edge_v, edge_attr, log_exposure,
           W1_l, b1_l, W1_r, W2_l, b2_l, W2_r, mW1, mb1, mW2, mb2):
    src = edge_index[0].astype(jnp.int32)
    dst = edge_index[1].astype(jnp.int32)
    u = edge_u.astype(jnp.int32)
    v = edge_v.astype(jnp.int32)
    zkd = jnp.zeros((KS, D), jnp.float32)
    onesd = jnp.ones((KS, D), jnp.float32)

    agg, deg = _agg_deg(x, src, dst, zkd, onesd)
    h1 = _tc_layer1(agg, deg, x, W1_l, b1_l.reshape(1, D), W1_r)
    (aggp2,) = _segsum(h1, src, dst, zkd)
    A, B = _tc_layer2(aggp2, deg, h1, W2_l, b2_l.reshape(1, D), W2_r,
                      mW1, mW1)
    (g,) = _gpair(A, B, u, v)
    out = _tc_edge(g, edge_attr, log_exposure.reshape(E // BE, RB, 128),
                   mW1[:, 2 * D:], mb1.reshape(1, D), mW2.reshape(D),
                   mb2)
    return out.reshape(E)
